# baseline (speedup 1.0000x reference)
import functools

import jax
import jax.numpy as jnp
from jax import lax
from jax.experimental import pallas as pl
from jax.experimental.pallas import tpu as pltpu
from jax.experimental.pallas import tpu_sc as plsc

ROW_BLK = 2000
EDGE_CHUNK = 400
NODE_CHUNK = 400
NS = 16
H_HALF = 16


def _proj_body(x_ref, wl_ref, bl_ref, wg_ref, za_ref, zb_ref):
    z1 = jnp.dot(x_ref[...], wl_ref[...], preferred_element_type=jnp.float32)
    z1 = z1 + bl_ref[...]
    z2 = jnp.dot(z1, wg_ref[...], preferred_element_type=jnp.float32)
    za_ref[...] = z2[:, :H_HALF]
    zb_ref[...] = z2[:, H_HALF:]


def _project(x, W_lin, b_lin, W_gc):
    n, d = x.shape
    h1 = W_lin.shape[1]
    h2 = W_gc.shape[1]
    grid = n // ROW_BLK
    return pl.pallas_call(
        _proj_body,
        grid=(grid,),
        in_specs=[
            pl.BlockSpec((ROW_BLK, d), lambda i: (i, 0)),
            pl.BlockSpec((d, h1), lambda i: (0, 0)),
            pl.BlockSpec((1, h1), lambda i: (0, 0)),
            pl.BlockSpec((h1, h2), lambda i: (0, 0)),
        ],
        out_specs=[
            pl.BlockSpec((ROW_BLK, H_HALF), lambda i: (i, 0)),
            pl.BlockSpec((ROW_BLK, H_HALF), lambda i: (i, 0)),
        ],
        out_shape=[
            jax.ShapeDtypeStruct((n, H_HALF), jnp.float32),
            jax.ShapeDtypeStruct((n, H_HALF), jnp.float32),
        ],
    )(x, W_lin, b_lin.reshape(1, h1), W_gc)


def _sc_aggregate(z2a, z2b, edge_index, b_gc):
    n = z2a.shape[0]
    e = edge_index.shape[1]
    h2 = b_gc.shape[0]
    ept = e // NS
    nchunks = ept // EDGE_CHUNK
    node_chunks = n // NODE_CHUNK
    ncpt = node_chunks // NS
    ncrem = node_chunks - ncpt * NS
    mesh = plsc.VectorSubcoreMesh(core_axis_name="c", subcore_axis_name="s")

    @functools.partial(
        pl.kernel,
        out_type=jax.ShapeDtypeStruct((n, h2), jnp.float32),
        mesh=mesh,
        compiler_params=pltpu.CompilerParams(use_tc_tiling_on_sc=False),
        scratch_types=[
            pltpu.VMEM_SHARED((n, H_HALF), jnp.float32),
            pltpu.VMEM_SHARED((n,), jnp.float32),
            pltpu.VMEM((EDGE_CHUNK,), jnp.int32),
            pltpu.VMEM((2, EDGE_CHUNK), jnp.int32),
            pltpu.VMEM((2, EDGE_CHUNK, H_HALF), jnp.float32),
            pltpu.VMEM((NODE_CHUNK, H_HALF), jnp.float32),
            pltpu.VMEM((EDGE_CHUNK,), jnp.float32),
            pltpu.VMEM((NODE_CHUNK,), jnp.float32),
            pltpu.VMEM((NODE_CHUNK,), jnp.float32),
            pltpu.VMEM((32,), jnp.float32),
            pltpu.SemaphoreType.DMA,
        ],
    )
    def body(za_hbm, zb_hbm, ei_hbm, bgc_hbm, out_hbm,
             agg_sh, deg_sh, srcb, dstb2, gbuf, rowsb, onesb, degb, recb, bgcb, sem):
        c = lax.axis_index("c")
        s = lax.axis_index("s")

        def fill16(i, carry):
            onesb[pl.ds(i * 16, 16)] = jnp.full((16,), 1.0, jnp.float32)
            degb[pl.ds(i * 16, 16)] = jnp.zeros((16,), jnp.float32)
            return carry
        lax.fori_loop(0, EDGE_CHUNK // 16, fill16, 0)

        def zero_rows(i, carry):
            rowsb[i] = jnp.zeros((H_HALF,), jnp.float32)
            return carry
        lax.fori_loop(0, NODE_CHUNK, zero_rows, 0)

        pltpu.sync_copy(bgc_hbm, bgcb)

        for j in range(ncpt):
            k = s + NS * j
            pltpu.sync_copy(rowsb, agg_sh.at[pl.ds(k * NODE_CHUNK, NODE_CHUNK)])
            pltpu.sync_copy(degb, deg_sh.at[pl.ds(k * NODE_CHUNK, NODE_CHUNK)])

        @pl.when(s < ncrem)
        def _zero_rem():
            k = ncpt * NS + s
            pltpu.sync_copy(rowsb, agg_sh.at[pl.ds(k * NODE_CHUNK, NODE_CHUNK)])
            pltpu.sync_copy(degb, deg_sh.at[pl.ds(k * NODE_CHUNK, NODE_CHUNK)])

        plsc.subcore_barrier()

        tile_base = s * ept

        def make_loop(table_hbm):
            def fetch(k, buf):
                eb = tile_base + k * EDGE_CHUNK
                pltpu.sync_copy(ei_hbm.at[0, pl.ds(eb, EDGE_CHUNK)], srcb)
                pltpu.sync_copy(ei_hbm.at[1, pl.ds(eb, EDGE_CHUNK)],
                                dstb2.at[buf])
                pltpu.make_async_copy(table_hbm.at[srcb], gbuf.at[buf],
                                      sem).start()

            def chunk(k, carry):
                buf = lax.rem(k, 2)
                nbuf = lax.rem(k + 1, 2)
                pltpu.make_async_copy(table_hbm.at[srcb], gbuf.at[buf],
                                      sem).wait()

                @pl.when(k + 1 < nchunks)
                def _():
                    fetch(k + 1, nbuf)
                pltpu.sync_copy(gbuf.at[buf], agg_sh.at[dstb2.at[buf]], add=True)
                pltpu.sync_copy(onesb, deg_sh.at[dstb2.at[buf]], add=True)
                return carry

            fetch(0, 0)
            lax.fori_loop(0, nchunks, chunk, 0)

        @pl.when(c == 0)
        def _loop_a():
            make_loop(za_hbm)

        @pl.when(c == 1)
        def _loop_b():
            make_loop(zb_hbm)

        plsc.subcore_barrier()

        bias_a = bgcb[pl.ds(0, H_HALF)]
        bias_b = bgcb[pl.ds(H_HALF, H_HALF)]

        def norm_chunk(k):
            r0 = k * NODE_CHUNK
            pltpu.sync_copy(agg_sh.at[pl.ds(r0, NODE_CHUNK)], rowsb)
            pltpu.sync_copy(deg_sh.at[pl.ds(r0, NODE_CHUNK)], degb)

            def recips(i, carry):
                d16 = degb[pl.ds(i * 16, 16)]
                recb[pl.ds(i * 16, 16)] = 1.0 / jnp.maximum(d16, 1.0)
                return carry
            lax.fori_loop(0, NODE_CHUNK // 16, recips, 0)

            def norm_rows(bias):
                def fn(i, carry):
                    rec16 = recb[pl.ds(i * 16, 16)]
                    base = i * 16
                    for j in range(16):
                        rowsb[base + j] = jnp.maximum(
                            rowsb[base + j] * rec16[j] + bias, 0.0)
                    return carry
                return fn

            @pl.when(c == 0)
            def _():
                lax.fori_loop(0, NODE_CHUNK // 16, norm_rows(bias_a), 0)
                pltpu.sync_copy(rowsb, out_hbm.at[pl.ds(r0, NODE_CHUNK), pl.ds(0, H_HALF)])

            @pl.when(c == 1)
            def _():
                lax.fori_loop(0, NODE_CHUNK // 16, norm_rows(bias_b), 0)
                pltpu.sync_copy(rowsb, out_hbm.at[pl.ds(r0, NODE_CHUNK), pl.ds(H_HALF, H_HALF)])

        for j in range(ncpt):
            norm_chunk(s + NS * j)

        @pl.when(s < ncrem)
        def _norm_rem():
            norm_chunk(ncpt * NS + s)

    return body(z2a, z2b, edge_index, b_gc)


def kernel(x, edge_index, W_lin, b_lin, W_gc, b_gc):
    ei = edge_index.astype(jnp.int32)
    z2a, z2b = _project(x, W_lin, b_lin, W_gc)
    return _sc_aggregate(z2a, z2b, ei, b_gc)


# speedup vs baseline: 1.1342x; 1.1342x over previous
import functools

import jax
import jax.numpy as jnp
from jax import lax
from jax.experimental import pallas as pl
from jax.experimental.pallas import tpu as pltpu
from jax.experimental.pallas import tpu_sc as plsc

ROW_BLK = 2000
EDGE_CHUNK = 800
NODE_CHUNK = 400
NS = 16
H_HALF = 16


def _proj_body(x_ref, wl_ref, bl_ref, wg_ref, za_ref, zb_ref):
    z1 = jnp.dot(x_ref[...], wl_ref[...], preferred_element_type=jnp.float32)
    z1 = z1 + bl_ref[...]
    z2 = jnp.dot(z1, wg_ref[...], preferred_element_type=jnp.float32)
    za_ref[...] = z2[:, :H_HALF]
    zb_ref[...] = z2[:, H_HALF:]


def _project(x, W_lin, b_lin, W_gc):
    n, d = x.shape
    h1 = W_lin.shape[1]
    h2 = W_gc.shape[1]
    grid = n // ROW_BLK
    return pl.pallas_call(
        _proj_body,
        grid=(grid,),
        in_specs=[
            pl.BlockSpec((ROW_BLK, d), lambda i: (i, 0)),
            pl.BlockSpec((d, h1), lambda i: (0, 0)),
            pl.BlockSpec((1, h1), lambda i: (0, 0)),
            pl.BlockSpec((h1, h2), lambda i: (0, 0)),
        ],
        out_specs=[
            pl.BlockSpec((ROW_BLK, H_HALF), lambda i: (i, 0)),
            pl.BlockSpec((ROW_BLK, H_HALF), lambda i: (i, 0)),
        ],
        out_shape=[
            jax.ShapeDtypeStruct((n, H_HALF), jnp.float32),
            jax.ShapeDtypeStruct((n, H_HALF), jnp.float32),
        ],
    )(x, W_lin, b_lin.reshape(1, h1), W_gc)


def _sc_aggregate(z2a, z2b, edge_index, b_gc):
    n = z2a.shape[0]
    e = edge_index.shape[1]
    h2 = b_gc.shape[0]
    ept = e // NS
    nchunks = ept // EDGE_CHUNK
    node_chunks = n // NODE_CHUNK
    ncpt = node_chunks // NS
    ncrem = node_chunks - ncpt * NS
    mesh = plsc.VectorSubcoreMesh(core_axis_name="c", subcore_axis_name="s")

    @functools.partial(
        pl.kernel,
        out_type=jax.ShapeDtypeStruct((n, h2), jnp.float32),
        mesh=mesh,
        compiler_params=pltpu.CompilerParams(use_tc_tiling_on_sc=False),
        scratch_types=[
            pltpu.VMEM_SHARED((n, H_HALF), jnp.float32),
            pltpu.VMEM_SHARED((n,), jnp.float32),
            pltpu.VMEM((EDGE_CHUNK,), jnp.int32),
            pltpu.VMEM((EDGE_CHUNK,), jnp.int32),
            pltpu.VMEM((EDGE_CHUNK, H_HALF), jnp.float32),
            pltpu.VMEM((NODE_CHUNK, H_HALF), jnp.float32),
            pltpu.VMEM((EDGE_CHUNK,), jnp.float32),
            pltpu.VMEM((NODE_CHUNK,), jnp.float32),
            pltpu.VMEM((NODE_CHUNK,), jnp.float32),
            pltpu.VMEM((32,), jnp.float32),
            pltpu.SemaphoreType.DMA,
        ],
    )
    def body(za_hbm, zb_hbm, ei_hbm, bgc_hbm, out_hbm,
             agg_sh, deg_sh, srcb, dstb, gbuf, rowsb, onesb, degb, recb, bgcb, sem):
        c = lax.axis_index("c")
        s = lax.axis_index("s")

        def fill16(i, carry):
            onesb[pl.ds(i * 16, 16)] = jnp.full((16,), 1.0, jnp.float32)
            degb[pl.ds(i * 16, 16)] = jnp.zeros((16,), jnp.float32)
            return carry
        lax.fori_loop(0, EDGE_CHUNK // 16, fill16, 0)

        def zero_rows(i, carry):
            rowsb[i] = jnp.zeros((H_HALF,), jnp.float32)
            return carry
        lax.fori_loop(0, NODE_CHUNK, zero_rows, 0)

        pltpu.sync_copy(bgc_hbm, bgcb)

        for j in range(ncpt):
            k = s + NS * j
            pltpu.sync_copy(rowsb, agg_sh.at[pl.ds(k * NODE_CHUNK, NODE_CHUNK)])
            pltpu.sync_copy(degb, deg_sh.at[pl.ds(k * NODE_CHUNK, NODE_CHUNK)])

        @pl.when(s < ncrem)
        def _zero_rem():
            k = ncpt * NS + s
            pltpu.sync_copy(rowsb, agg_sh.at[pl.ds(k * NODE_CHUNK, NODE_CHUNK)])
            pltpu.sync_copy(degb, deg_sh.at[pl.ds(k * NODE_CHUNK, NODE_CHUNK)])

        plsc.subcore_barrier()

        tile_base = s * ept

        def make_loop(table_hbm):
            def chunk(k, carry):
                eb = tile_base + k * EDGE_CHUNK
                pltpu.sync_copy(ei_hbm.at[0, pl.ds(eb, EDGE_CHUNK)], srcb)
                pltpu.sync_copy(ei_hbm.at[1, pl.ds(eb, EDGE_CHUNK)], dstb)
                pltpu.async_copy(table_hbm.at[srcb], gbuf, sem).wait()
                pltpu.sync_copy(gbuf, agg_sh.at[dstb], add=True)
                pltpu.sync_copy(onesb, deg_sh.at[dstb], add=True)
                return carry

            lax.fori_loop(0, nchunks, chunk, 0)

        @pl.when(c == 0)
        def _loop_a():
            make_loop(za_hbm)

        @pl.when(c == 1)
        def _loop_b():
            make_loop(zb_hbm)

        plsc.subcore_barrier()

        bias_a = bgcb[pl.ds(0, H_HALF)]
        bias_b = bgcb[pl.ds(H_HALF, H_HALF)]

        def norm_chunk(k):
            r0 = k * NODE_CHUNK
            pltpu.sync_copy(agg_sh.at[pl.ds(r0, NODE_CHUNK)], rowsb)
            pltpu.sync_copy(deg_sh.at[pl.ds(r0, NODE_CHUNK)], degb)

            def recips(i, carry):
                d16 = degb[pl.ds(i * 16, 16)]
                recb[pl.ds(i * 16, 16)] = 1.0 / jnp.maximum(d16, 1.0)
                return carry
            lax.fori_loop(0, NODE_CHUNK // 16, recips, 0)

            def norm_rows(bias):
                def fn(i, carry):
                    rec16 = recb[pl.ds(i * 16, 16)]
                    base = i * 16
                    for j in range(16):
                        rowsb[base + j] = jnp.maximum(
                            rowsb[base + j] * rec16[j] + bias, 0.0)
                    return carry
                return fn

            @pl.when(c == 0)
            def _():
                lax.fori_loop(0, NODE_CHUNK // 16, norm_rows(bias_a), 0)
                pltpu.sync_copy(rowsb, out_hbm.at[pl.ds(r0, NODE_CHUNK), pl.ds(0, H_HALF)])

            @pl.when(c == 1)
            def _():
                lax.fori_loop(0, NODE_CHUNK // 16, norm_rows(bias_b), 0)
                pltpu.sync_copy(rowsb, out_hbm.at[pl.ds(r0, NODE_CHUNK), pl.ds(H_HALF, H_HALF)])

        for j in range(ncpt):
            norm_chunk(s + NS * j)

        @pl.when(s < ncrem)
        def _norm_rem():
            norm_chunk(ncpt * NS + s)

    return body(z2a, z2b, edge_index, b_gc)


def kernel(x, edge_index, W_lin, b_lin, W_gc, b_gc):
    ei = edge_index.astype(jnp.int32)
    z2a, z2b = _project(x, W_lin, b_lin, W_gc)
    return _sc_aggregate(z2a, z2b, ei, b_gc)
